# baseline (speedup 1.0000x reference)
import jax
import jax.numpy as jnp
from jax import lax
from jax.experimental import pallas as pl
from jax.experimental.pallas import tpu as pltpu
from jax.experimental.pallas import tpu_sc as plsc

N = 10000
E = 160000
D = 128
H = 8
DH = 16
B = 1024
OUT = 64
TAU = 0.8
LAM = 0.5

ROWS = 400
NBLK = N // ROWS
CH = 128
CH2 = CH // 2
NCHUNK = E // CH
NTILES = 32
NST = NCHUNK // NTILES
NPAD = 10112
ROWS_PER_TILE = NPAD // 16
SBLK = 1024
NSBLK = (N + SBLK - 1) // SBLK


def _elu(x):
    return jnp.where(x > 0, x, jnp.exp(x) - 1.0)


def _k1_body(feat_r, posf_r, wfc_r, bfc_r, wsc_r, a2_r, ad2_r,
             h_r, ph_r, hw_r, es2_r, ed2_r, mges_r, mged_r):
    i = pl.program_id(0)
    wfc = wfc_r[...]
    b = bfc_r[...]
    h = _elu(jnp.dot(feat_r[...], wfc, preferred_element_type=jnp.float32) + b)
    ph = _elu(jnp.dot(posf_r[...], wfc, preferred_element_type=jnp.float32) + b)
    hw = jnp.dot(h, wsc_r[...], preferred_element_type=jnp.float32)
    es2 = jnp.dot(hw, a2_r[...], preferred_element_type=jnp.float32)
    ed2 = jnp.dot(hw, ad2_r[...], preferred_element_type=jnp.float32)
    h_r[...] = h
    ph_r[...] = ph
    hw_r[...] = hw
    es2_r[...] = es2
    ed2_r[...] = ed2

    @pl.when(i == 0)
    def _():
        mges_r[...] = jnp.full((1, 16), -1e30, jnp.float32)
        mged_r[...] = jnp.full((1, 16), -1e30, jnp.float32)

    mxs = jnp.max(es2, axis=0, keepdims=True)
    mxd = jnp.max(ed2, axis=0, keepdims=True)
    mges_r[...] = jnp.maximum(mges_r[...],
                              jnp.concatenate([mxs, mxs], axis=1))
    mged_r[...] = jnp.maximum(mged_r[...],
                              jnp.concatenate([mxd, mxd], axis=1))


def _k1(feat, pos_feat, w_fc, b_fc, w_sc, a2, ad2):
    f32 = jnp.float32
    return pl.pallas_call(
        _k1_body,
        grid=(NBLK,),
        in_specs=[
            pl.BlockSpec((ROWS, D), lambda i: (i, 0)),
            pl.BlockSpec((ROWS, D), lambda i: (i, 0)),
            pl.BlockSpec((D, D), lambda i: (0, 0)),
            pl.BlockSpec((1, D), lambda i: (0, 0)),
            pl.BlockSpec((D, D), lambda i: (0, 0)),
            pl.BlockSpec((D, H), lambda i: (0, 0)),
            pl.BlockSpec((D, H), lambda i: (0, 0)),
        ],
        out_specs=[
            pl.BlockSpec((ROWS, D), lambda i: (i, 0)),
            pl.BlockSpec((ROWS, D), lambda i: (i, 0)),
            pl.BlockSpec((ROWS, D), lambda i: (i, 0)),
            pl.BlockSpec((ROWS, H), lambda i: (i, 0)),
            pl.BlockSpec((ROWS, H), lambda i: (i, 0)),
            pl.BlockSpec((1, 16), lambda i: (0, 0)),
            pl.BlockSpec((1, 16), lambda i: (0, 0)),
        ],
        out_shape=[
            jax.ShapeDtypeStruct((N, D), f32),
            jax.ShapeDtypeStruct((N, D), f32),
            jax.ShapeDtypeStruct((N, D), f32),
            jax.ShapeDtypeStruct((N, H), f32),
            jax.ShapeDtypeStruct((N, H), f32),
            jax.ShapeDtypeStruct((1, 16), f32),
            jax.ShapeDtypeStruct((1, 16), f32),
        ],
    )(feat, pos_feat, w_fc, b_fc, w_sc, a2, ad2)


def _sc_edges_body(hw_hbm, es2_hbm, ed2_hbm, mges_hbm, mged_hbm,
                   src_hbm, dst_hbm, psrc_hbm, pdst_hbm, ph_hbm,
                   z128_hbm, z8_hbm, ones_hbm,
                   num_out, den_out, agg_out, deg_out,
                   srcv, dstv, av, bv, rv, exv, mg1v, mg2v, onev,
                   big_sh, small_sh, isem, gsem, ssem):
    c = lax.axis_index("c")
    s = lax.axis_index("s")
    wid = s * 2 + c
    base = s * ROWS_PER_TILE

    pltpu.sync_copy(mges_hbm, mg1v)
    pltpu.sync_copy(mged_hbm, mg2v)
    pltpu.sync_copy(ones_hbm, onev)
    mg2 = jnp.maximum(mg1v[...] + mg2v[...], 0.0)
    iota16 = lax.iota(jnp.int32, 16)
    rowpat = lax.shift_right_logical(iota16, 3)
    colpat = lax.bitwise_and(iota16, 7)

    def zero_acc():
        pltpu.sync_copy(z128_hbm, big_sh.at[pl.ds(base, ROWS_PER_TILE)])
        pltpu.sync_copy(z8_hbm, small_sh.at[pl.ds(base, ROWS_PER_TILE)])

    def copy_out(big_o, small_o):
        pltpu.sync_copy(big_sh.at[pl.ds(base, ROWS_PER_TILE)],
                        big_o.at[c, pl.ds(base, ROWS_PER_TILE)])
        pltpu.sync_copy(small_sh.at[pl.ds(base, ROWS_PER_TILE)],
                        small_o.at[c, pl.ds(base, ROWS_PER_TILE)])

    def compute(b):
        def edge_body(j, _):
            rix = 2 * j + rowpat
            a16 = plsc.load_gather(av.at[b], [rix, colpat])
            b16 = plsc.load_gather(bv.at[b], [rix, colpat])
            e = a16 + b16
            e = jnp.where(e > 0, e, 0.2 * e)
            ex = jnp.exp(e - mg2)
            plsc.store_scatter(exv.at[b], [rix, colpat], ex)
            i0 = 2 * j
            i1 = 2 * j + 1
            for hh in range(H):
                sl = pl.ds(hh * DH, DH)
                rv[b, i0, sl] = rv[b, i0, sl] * ex[hh]
                rv[b, i1, sl] = rv[b, i1, sl] * ex[8 + hh]
            return 0

        lax.fori_loop(0, CH2, edge_body, 0)

    def run_phase(sa_hbm, da_hbm, table_hbm, att):
        def off_of(k):
            return jnp.where(k < NST, (k * NTILES + wid) * CH,
                             (NST * NTILES + wid) * CH)

        def start_idx(k, i):
            off = off_of(k)
            pltpu.async_copy(sa_hbm.at[pl.ds(off, CH)], srcv.at[i, 0],
                             isem.at[i, 0])
            pltpu.async_copy(da_hbm.at[pl.ds(off, CH)], dstv.at[i, 0],
                             isem.at[i, 1])

        def wait_idx(k, i):
            off = off_of(k)
            pltpu.make_async_copy(sa_hbm.at[pl.ds(off, CH)], srcv.at[i, 0],
                                  isem.at[i, 0]).wait()
            pltpu.make_async_copy(da_hbm.at[pl.ds(off, CH)], dstv.at[i, 0],
                                  isem.at[i, 1]).wait()

        def start_gathers(b, i):
            pltpu.async_copy(table_hbm.at[srcv.at[i, 0]], rv.at[b],
                             gsem.at[b, 0])
            if att:
                pltpu.async_copy(es2_hbm.at[srcv.at[i, 0]], av.at[b],
                                 gsem.at[b, 1])
                pltpu.async_copy(ed2_hbm.at[dstv.at[i, 0]], bv.at[b],
                                 gsem.at[b, 2])

        def wait_gathers(b, i):
            pltpu.make_async_copy(table_hbm.at[srcv.at[i, 0]], rv.at[b],
                                  gsem.at[b, 0]).wait()
            if att:
                pltpu.make_async_copy(es2_hbm.at[srcv.at[i, 0]], av.at[b],
                                      gsem.at[b, 1]).wait()
                pltpu.make_async_copy(ed2_hbm.at[dstv.at[i, 0]], bv.at[b],
                                      gsem.at[b, 2]).wait()

        def start_scatters(b, i):
            pltpu.async_copy(rv.at[b], big_sh.at[dstv.at[i, 0]],
                             ssem.at[b, 0], add=True)
            small_src = exv.at[b] if att else onev
            pltpu.async_copy(small_src, small_sh.at[dstv.at[i, 0]],
                             ssem.at[b, 1], add=True)

        def wait_scatters(b, i):
            pltpu.make_async_copy(rv.at[b], big_sh.at[dstv.at[i, 0]],
                                  ssem.at[b, 0]).wait()
            small_src = exv.at[b] if att else onev
            pltpu.make_async_copy(small_src, small_sh.at[dstv.at[i, 0]],
                                  ssem.at[b, 1]).wait()

        nst_i = NST + jnp.where(wid < NCHUNK % NTILES, 1, 0)

        start_idx(0, 0)
        start_idx(1, 1)
        wait_idx(0, 0)
        start_gathers(0, 0)

        def chunk_body(k, _):
            b = lax.rem(k, 2)
            nb = 1 - b
            ik = lax.rem(k, 3)
            i1 = lax.rem(k + 1, 3)
            i2 = lax.rem(k + 2, 3)
            wait_gathers(b, ik)

            @pl.when(k >= 1)
            def _():
                wait_scatters(nb, i2)

            @pl.when(k + 1 < nst_i)
            def _():
                wait_idx(k + 1, i1)
                start_gathers(nb, i1)

            @pl.when(k + 2 < nst_i)
            def _():
                start_idx(k + 2, i2)

            if att:
                compute(b)
            start_scatters(b, ik)
            return 0

        lax.fori_loop(0, nst_i, chunk_body, 0)
        wait_scatters(lax.rem(nst_i - 1, 2), lax.rem(nst_i - 1, 3))

    zero_acc()
    plsc.subcore_barrier()
    run_phase(src_hbm, dst_hbm, hw_hbm, True)
    plsc.subcore_barrier()
    copy_out(num_out, den_out)
    zero_acc()
    plsc.subcore_barrier()
    run_phase(psrc_hbm, pdst_hbm, ph_hbm, False)
    plsc.subcore_barrier()
    copy_out(agg_out, deg_out)


def _sc_edges(hw, es2, ed2, mges, mged, src, dst, psrc, pdst, ph,
              z128, z8, ones8):
    f32 = jnp.float32
    mesh = plsc.VectorSubcoreMesh(core_axis_name="c", subcore_axis_name="s")
    call = pl.kernel(
        _sc_edges_body,
        out_type=[
            jax.ShapeDtypeStruct((2, NPAD, D), f32),
            jax.ShapeDtypeStruct((2, NPAD, H), f32),
            jax.ShapeDtypeStruct((2, NPAD, D), f32),
            jax.ShapeDtypeStruct((2, NPAD, H), f32),
        ],
        mesh=mesh,
        compiler_params=pltpu.CompilerParams(use_tc_tiling_on_sc=False,
                                             needs_layout_passes=False),
        scratch_types=[
            pltpu.VMEM((3, 1, CH), jnp.int32),
            pltpu.VMEM((3, 1, CH), jnp.int32),
            pltpu.VMEM((2, CH, H), f32),
            pltpu.VMEM((2, CH, H), f32),
            pltpu.VMEM((2, CH, D), f32),
            pltpu.VMEM((2, CH, H), f32),
            pltpu.VMEM((16,), f32),
            pltpu.VMEM((16,), f32),
            pltpu.VMEM((CH, H), f32),
            pltpu.VMEM_SHARED((NPAD, D), f32),
            pltpu.VMEM_SHARED((NPAD, H), f32),
            pltpu.SemaphoreType.DMA((3, 2)),
            pltpu.SemaphoreType.DMA((2, 3)),
            pltpu.SemaphoreType.DMA((2, 2)),
        ],
    )
    return call(hw, es2, ed2, mges.reshape(16), mged.reshape(16),
                src, dst, psrc, pdst, ph, z128, z8, ones8)


def _k2_body(np_r, dp_r, h_r, ap_r, gp_r, wmp_r, pa_r,
             wp1_r, bp1_r, wp2_r, bp2_r, m_r,
             zsn_r, zmn_r, zsc_r):
    m = m_r[...]
    num = np_r[0] + np_r[1]
    den = dp_r[0] + dp_r[1]
    rden = 1.0 / (den + 1e-9)
    zsc = _elu(num * jnp.dot(rden, m, preferred_element_type=jnp.float32)
               + h_r[...])
    agg = ap_r[0] + ap_r[1]
    deg = gp_r[0] + gp_r[1]
    invd = 1.0 / jnp.maximum(deg, 1.0)
    aggn = agg * jnp.dot(invd, m, preferred_element_type=jnp.float32)
    zmp = jnp.dot(aggn, wmp_r[...], preferred_element_type=jnp.float32)
    pa = pa_r[0, 0]
    zmp = jnp.where(zmp > 0, zmp, pa * zmp)

    wp1 = wp1_r[...]
    bp1 = bp1_r[...]
    wp2 = wp2_r[...]
    bp2 = bp2_r[...]

    def proj(z):
        t = _elu(jnp.dot(z, wp1, preferred_element_type=jnp.float32) + bp1)
        return jnp.dot(t, wp2, preferred_element_type=jnp.float32) + bp2

    zs = proj(zsc)
    zm = proj(zmp)
    zsn = zs / (jnp.sqrt(jnp.sum(zs * zs, axis=1, keepdims=True)) + 1e-8)
    zmn = zm / (jnp.sqrt(jnp.sum(zm * zm, axis=1, keepdims=True)) + 1e-8)
    zsn_r[...] = zsn
    zmn_r[...] = zmn
    zsc_r[...] = zsc


def _k2(num_p, den_p, h, agg_p, deg_p, w_mp, pa, wp1, bp1, wp2, bp2, m):
    f32 = jnp.float32
    return pl.pallas_call(
        _k2_body,
        grid=(NBLK,),
        in_specs=[
            pl.BlockSpec((2, ROWS, D), lambda i: (0, i, 0)),
            pl.BlockSpec((2, ROWS, H), lambda i: (0, i, 0)),
            pl.BlockSpec((ROWS, D), lambda i: (i, 0)),
            pl.BlockSpec((2, ROWS, D), lambda i: (0, i, 0)),
            pl.BlockSpec((2, ROWS, H), lambda i: (0, i, 0)),
            pl.BlockSpec((D, D), lambda i: (0, 0)),
            pl.BlockSpec((1, 1), lambda i: (0, 0)),
            pl.BlockSpec((D, D), lambda i: (0, 0)),
            pl.BlockSpec((1, D), lambda i: (0, 0)),
            pl.BlockSpec((D, D), lambda i: (0, 0)),
            pl.BlockSpec((1, D), lambda i: (0, 0)),
            pl.BlockSpec((H, D), lambda i: (0, 0)),
        ],
        out_specs=[
            pl.BlockSpec((ROWS, D), lambda i: (i, 0)),
            pl.BlockSpec((ROWS, D), lambda i: (i, 0)),
            pl.BlockSpec((ROWS, D), lambda i: (i, 0)),
        ],
        out_shape=[
            jax.ShapeDtypeStruct((N, D), f32),
            jax.ShapeDtypeStruct((N, D), f32),
            jax.ShapeDtypeStruct((N, D), f32),
        ],
    )(num_p, den_p, h, agg_p, deg_p, w_mp, pa, wp1, bp1, wp2, bp2, m)


def _k3_body(zsnb_r, zmnb_r, zsn_r, zmn_r, pos_r, zscb_r, wp_r, bp_r,
             sums_r, loss_r, out_r):
    i = pl.program_id(0)

    @pl.when(i == 0)
    def _():
        sums_r[...] = jnp.zeros((B, 4), jnp.float32)
        loss_r[...] = jnp.zeros((1, 1), jnp.float32)
        out_r[...] = (jnp.dot(zscb_r[...], wp_r[...],
                              preferred_element_type=jnp.float32) + bp_r[...])

    zsnb = zsnb_r[...]
    zmnb = zmnb_r[...]
    zsn = zsn_r[...]
    zmn = zmn_r[...]
    posf = pos_r[...].astype(jnp.float32)
    col = lax.broadcasted_iota(jnp.int32, (B, SBLK), 1) + i * SBLK
    valid = col < N
    dn = (((1,), (1,)), ((), ()))
    ssc = jnp.exp(lax.dot_general(zsnb, zmn, dn,
                                  preferred_element_type=jnp.float32) / TAU)
    ssc = jnp.where(valid, ssc, 0.0)
    smp = jnp.exp(lax.dot_general(zmnb, zsn, dn,
                                  preferred_element_type=jnp.float32) / TAU)
    smp = jnp.where(valid, smp, 0.0)
    contrib = jnp.concatenate([
        jnp.sum(ssc * posf, axis=1, keepdims=True),
        jnp.sum(ssc, axis=1, keepdims=True),
        jnp.sum(smp * posf, axis=1, keepdims=True),
        jnp.sum(smp, axis=1, keepdims=True),
    ], axis=1)
    sums_r[...] = sums_r[...] + contrib

    @pl.when(i == NSBLK - 1)
    def _():
        v = sums_r[...]
        lsc = -jnp.log(v[:, 0:1] / (v[:, 1:2] + 1e-9) + 1e-9)
        lmp = -jnp.log(v[:, 2:3] / (v[:, 3:4] + 1e-9) + 1e-9)
        msc = jnp.sum(lsc, axis=0, keepdims=True) / B
        mmp = jnp.sum(lmp, axis=0, keepdims=True) / B
        loss_r[...] = LAM * msc + (1.0 - LAM) * mmp


def _k3(zsn, zmn, pos, zsc, w_pred, b_pred):
    f32 = jnp.float32
    return pl.pallas_call(
        _k3_body,
        grid=(NSBLK,),
        in_specs=[
            pl.BlockSpec((B, D), lambda i: (0, 0)),
            pl.BlockSpec((B, D), lambda i: (0, 0)),
            pl.BlockSpec((SBLK, D), lambda i: (i, 0)),
            pl.BlockSpec((SBLK, D), lambda i: (i, 0)),
            pl.BlockSpec((B, SBLK), lambda i: (0, i)),
            pl.BlockSpec((B, D), lambda i: (0, 0)),
            pl.BlockSpec((D, OUT), lambda i: (0, 0)),
            pl.BlockSpec((1, OUT), lambda i: (0, 0)),
        ],
        out_specs=[
            pl.BlockSpec((B, 4), lambda i: (0, 0)),
            pl.BlockSpec((1, 1), lambda i: (0, 0)),
            pl.BlockSpec((B, OUT), lambda i: (0, 0)),
        ],
        out_shape=[
            jax.ShapeDtypeStruct((B, 4), f32),
            jax.ShapeDtypeStruct((1, 1), f32),
            jax.ShapeDtypeStruct((B, OUT), f32),
        ],
    )(zsn, zmn, zsn, zmn, pos, zsc, w_pred, b_pred)


def kernel(feat, edge_index, pos_feat, pos_edge_index, pos,
           W_fc, b_fc, W_sc, a_src, a_dst, W_mp, prelu_a,
           Wp1, bp1, Wp2, bp2, W_pred, b_pred):
    f32 = jnp.float32
    headsel = (jnp.arange(D)[:, None] // DH) == jnp.arange(H)[None, :]
    a2 = jnp.where(headsel, a_src.reshape(D)[:, None], 0.0)
    ad2 = jnp.where(headsel, a_dst.reshape(D)[:, None], 0.0)
    m16 = jnp.where(
        (jnp.arange(D)[None, :] // DH) == jnp.arange(H)[:, None],
        1.0, 0.0).astype(f32)

    h, ph, hw, es2, ed2, mges, mged = _k1(
        feat, pos_feat, W_fc, b_fc.reshape(1, D), W_sc, a2, ad2)

    z128 = jnp.zeros((ROWS_PER_TILE, D), f32)
    z8 = jnp.zeros((ROWS_PER_TILE, H), f32)
    ones8 = jnp.ones((CH, H), f32)

    num_p, den_p, agg_p, deg_p = _sc_edges(
        hw, es2, ed2, mges, mged, edge_index[0], edge_index[1],
        pos_edge_index[0], pos_edge_index[1], ph, z128, z8, ones8)

    zsn, zmn, zsc = _k2(num_p, den_p, h, agg_p, deg_p, W_mp,
                        prelu_a.reshape(1, 1), Wp1, bp1.reshape(1, D),
                        Wp2, bp2.reshape(1, D), m16)

    pos8 = pos.view(jnp.int8)
    _, loss2d, out = _k3(zsn, zmn, pos8, zsc, W_pred, b_pred.reshape(1, OUT))
    return loss2d.reshape(()), out


# speedup vs baseline: 1.0344x; 1.0344x over previous
import jax
import jax.numpy as jnp
from jax import lax
from jax.experimental import pallas as pl
from jax.experimental.pallas import tpu as pltpu
from jax.experimental.pallas import tpu_sc as plsc

N = 10000
E = 160000
D = 128
H = 8
DH = 16
B = 1024
OUT = 64
TAU = 0.8
LAM = 0.5

ROWS = 400
NBLK = N // ROWS
CH = 128
CH2 = CH // 2
NCHUNK = E // CH
NTILES = 32
NST = NCHUNK // NTILES
NPAD = 10112
ROWS_PER_TILE = NPAD // 16
SBLK = 1024
NSBLK = (N + SBLK - 1) // SBLK


def _elu(x):
    return jnp.where(x > 0, x, jnp.exp(x) - 1.0)


def _k1_body(feat_r, posf_r, wfc_r, bfc_r, wsc_r, a2_r, ad2_r,
             h_r, ph_r, hw_r, es2_r, ed2_r, mges_r, mged_r):
    i = pl.program_id(0)
    wfc = wfc_r[...]
    b = bfc_r[...]
    h = _elu(jnp.dot(feat_r[...], wfc, preferred_element_type=jnp.float32) + b)
    ph = _elu(jnp.dot(posf_r[...], wfc, preferred_element_type=jnp.float32) + b)
    hw = jnp.dot(h, wsc_r[...], preferred_element_type=jnp.float32)
    es2 = jnp.dot(hw, a2_r[...], preferred_element_type=jnp.float32)
    ed2 = jnp.dot(hw, ad2_r[...], preferred_element_type=jnp.float32)
    h_r[...] = h
    ph_r[...] = ph
    hw_r[...] = hw
    es2_r[...] = es2
    ed2_r[...] = ed2

    @pl.when(i == 0)
    def _():
        mges_r[...] = jnp.full((1, 16), -1e30, jnp.float32)
        mged_r[...] = jnp.full((1, 16), -1e30, jnp.float32)

    mxs = jnp.max(es2, axis=0, keepdims=True)
    mxd = jnp.max(ed2, axis=0, keepdims=True)
    mges_r[...] = jnp.maximum(mges_r[...],
                              jnp.concatenate([mxs, mxs], axis=1))
    mged_r[...] = jnp.maximum(mged_r[...],
                              jnp.concatenate([mxd, mxd], axis=1))


def _k1(feat, pos_feat, w_fc, b_fc, w_sc, a2, ad2):
    f32 = jnp.float32
    return pl.pallas_call(
        _k1_body,
        grid=(NBLK,),
        in_specs=[
            pl.BlockSpec((ROWS, D), lambda i: (i, 0)),
            pl.BlockSpec((ROWS, D), lambda i: (i, 0)),
            pl.BlockSpec((D, D), lambda i: (0, 0)),
            pl.BlockSpec((1, D), lambda i: (0, 0)),
            pl.BlockSpec((D, D), lambda i: (0, 0)),
            pl.BlockSpec((D, H), lambda i: (0, 0)),
            pl.BlockSpec((D, H), lambda i: (0, 0)),
        ],
        out_specs=[
            pl.BlockSpec((ROWS, D), lambda i: (i, 0)),
            pl.BlockSpec((ROWS, D), lambda i: (i, 0)),
            pl.BlockSpec((ROWS, D), lambda i: (i, 0)),
            pl.BlockSpec((ROWS, H), lambda i: (i, 0)),
            pl.BlockSpec((ROWS, H), lambda i: (i, 0)),
            pl.BlockSpec((1, 16), lambda i: (0, 0)),
            pl.BlockSpec((1, 16), lambda i: (0, 0)),
        ],
        out_shape=[
            jax.ShapeDtypeStruct((N, D), f32),
            jax.ShapeDtypeStruct((N, D), f32),
            jax.ShapeDtypeStruct((N, D), f32),
            jax.ShapeDtypeStruct((N, H), f32),
            jax.ShapeDtypeStruct((N, H), f32),
            jax.ShapeDtypeStruct((1, 16), f32),
            jax.ShapeDtypeStruct((1, 16), f32),
        ],
    )(feat, pos_feat, w_fc, b_fc, w_sc, a2, ad2)


def _sc_edges_body(hw_hbm, es2_hbm, ed2_hbm, mges_hbm, mged_hbm,
                   src_hbm, dst_hbm, psrc_hbm, pdst_hbm, ph_hbm,
                   z128_hbm, z8_hbm, ones_hbm,
                   num_out, den_out, agg_out, deg_out,
                   srcv, dstv, av, bv, rv, exv, mg1v, mg2v, onev,
                   big_sh, small_sh, isem, gsem, ssem):
    c = lax.axis_index("c")
    s = lax.axis_index("s")
    wid = s * 2 + c
    base = s * ROWS_PER_TILE

    pltpu.sync_copy(mges_hbm, mg1v)
    pltpu.sync_copy(mged_hbm, mg2v)
    pltpu.sync_copy(ones_hbm, onev)
    mg2 = jnp.maximum(mg1v[...] + mg2v[...], 0.0)
    iota16 = lax.iota(jnp.int32, 16)
    rowpat = lax.shift_right_logical(iota16, 3)
    colpat = lax.bitwise_and(iota16, 7)

    def zero_acc():
        pltpu.sync_copy(z128_hbm, big_sh.at[pl.ds(base, ROWS_PER_TILE)])
        pltpu.sync_copy(z8_hbm, small_sh.at[pl.ds(base, ROWS_PER_TILE)])

    def copy_out(big_o, small_o):
        pltpu.sync_copy(big_sh.at[pl.ds(base, ROWS_PER_TILE)],
                        big_o.at[c, pl.ds(base, ROWS_PER_TILE)])
        pltpu.sync_copy(small_sh.at[pl.ds(base, ROWS_PER_TILE)],
                        small_o.at[c, pl.ds(base, ROWS_PER_TILE)])

    def compute(b):
        def edge_body(j, _):
            rix = 2 * j + rowpat
            a16 = plsc.load_gather(av.at[b], [rix, colpat])
            b16 = plsc.load_gather(bv.at[b], [rix, colpat])
            e = a16 + b16
            e = jnp.where(e > 0, e, 0.2 * e)
            ex = jnp.exp(e - mg2)
            plsc.store_scatter(exv.at[b], [rix, colpat], ex)
            i0 = 2 * j
            i1 = 2 * j + 1
            for hh in range(H):
                sl = pl.ds(hh * DH, DH)
                rv[b, i0, sl] = rv[b, i0, sl] * ex[hh]
                rv[b, i1, sl] = rv[b, i1, sl] * ex[8 + hh]
            return 0

        lax.fori_loop(0, CH2, edge_body, 0)

    def run_phase(sa_hbm, da_hbm, table_hbm, att):
        def off_of(k):
            return jnp.where(k < NST, (k * NTILES + wid) * CH,
                             (NST * NTILES + wid) * CH)

        def start_idx(k, i):
            off = off_of(k)
            pltpu.async_copy(sa_hbm.at[pl.ds(off, CH)], srcv.at[i, 0],
                             isem.at[i, 0])
            pltpu.async_copy(da_hbm.at[pl.ds(off, CH)], dstv.at[i, 0],
                             isem.at[i, 1])

        def wait_idx(k, i):
            off = off_of(k)
            pltpu.make_async_copy(sa_hbm.at[pl.ds(off, CH)], srcv.at[i, 0],
                                  isem.at[i, 0]).wait()
            pltpu.make_async_copy(da_hbm.at[pl.ds(off, CH)], dstv.at[i, 0],
                                  isem.at[i, 1]).wait()

        def start_gathers(b, i):
            pltpu.async_copy(table_hbm.at[srcv.at[i, 0]], rv.at[b],
                             gsem.at[b, 0])
            if att:
                pltpu.async_copy(es2_hbm.at[srcv.at[i, 0]], av.at[b],
                                 gsem.at[b, 1])
                pltpu.async_copy(ed2_hbm.at[dstv.at[i, 0]], bv.at[b],
                                 gsem.at[b, 2])

        def wait_gathers(b, i):
            pltpu.make_async_copy(table_hbm.at[srcv.at[i, 0]], rv.at[b],
                                  gsem.at[b, 0]).wait()
            if att:
                pltpu.make_async_copy(es2_hbm.at[srcv.at[i, 0]], av.at[b],
                                      gsem.at[b, 1]).wait()
                pltpu.make_async_copy(ed2_hbm.at[dstv.at[i, 0]], bv.at[b],
                                      gsem.at[b, 2]).wait()

        def start_scatters(b, i):
            pltpu.async_copy(rv.at[b], big_sh.at[dstv.at[i, 0]],
                             ssem.at[b, 0], add=True)
            small_src = exv.at[b] if att else onev
            pltpu.async_copy(small_src, small_sh.at[dstv.at[i, 0]],
                             ssem.at[b, 1], add=True)

        def wait_scatters(b, i):
            pltpu.make_async_copy(rv.at[b], big_sh.at[dstv.at[i, 0]],
                                  ssem.at[b, 0]).wait()
            small_src = exv.at[b] if att else onev
            pltpu.make_async_copy(small_src, small_sh.at[dstv.at[i, 0]],
                                  ssem.at[b, 1]).wait()

        nst_i = NST + jnp.where(wid < NCHUNK % NTILES, 1, 0)

        start_idx(0, 0)
        start_idx(1, 1)
        wait_idx(0, 0)
        start_gathers(0, 0)

        def chunk_body(k, _):
            b = lax.rem(k, 2)
            nb = 1 - b
            ik = lax.rem(k, 3)
            i1 = lax.rem(k + 1, 3)
            i2 = lax.rem(k + 2, 3)
            wait_gathers(b, ik)

            @pl.when(k >= 1)
            def _():
                wait_scatters(nb, i2)

            @pl.when(k + 1 < nst_i)
            def _():
                wait_idx(k + 1, i1)
                start_gathers(nb, i1)

            @pl.when(k + 2 < nst_i)
            def _():
                start_idx(k + 2, i2)

            if att:
                compute(b)
            start_scatters(b, ik)
            return 0

        lax.fori_loop(0, nst_i, chunk_body, 0)
        wait_scatters(lax.rem(nst_i - 1, 2), lax.rem(nst_i - 1, 3))

    zero_acc()
    plsc.subcore_barrier()
    run_phase(src_hbm, dst_hbm, hw_hbm, True)
    plsc.subcore_barrier()
    copy_out(num_out, den_out)
    zero_acc()
    plsc.subcore_barrier()
    run_phase(psrc_hbm, pdst_hbm, ph_hbm, False)
    plsc.subcore_barrier()
    copy_out(agg_out, deg_out)


def _sc_edges(hw, es2, ed2, mges, mged, src, dst, psrc, pdst, ph,
              z128, z8, ones8):
    f32 = jnp.float32
    mesh = plsc.VectorSubcoreMesh(core_axis_name="c", subcore_axis_name="s")
    call = pl.kernel(
        _sc_edges_body,
        out_type=[
            jax.ShapeDtypeStruct((2, NPAD, D), f32),
            jax.ShapeDtypeStruct((2, NPAD, H), f32),
            jax.ShapeDtypeStruct((2, NPAD, D), f32),
            jax.ShapeDtypeStruct((2, NPAD, H), f32),
        ],
        mesh=mesh,
        compiler_params=pltpu.CompilerParams(use_tc_tiling_on_sc=False,
                                             needs_layout_passes=False),
        scratch_types=[
            pltpu.VMEM((3, 1, CH), jnp.int32),
            pltpu.VMEM((3, 1, CH), jnp.int32),
            pltpu.VMEM((2, CH, H), f32),
            pltpu.VMEM((2, CH, H), f32),
            pltpu.VMEM((2, CH, D), f32),
            pltpu.VMEM((2, CH, H), f32),
            pltpu.VMEM((16,), f32),
            pltpu.VMEM((16,), f32),
            pltpu.VMEM((CH, H), f32),
            pltpu.VMEM_SHARED((NPAD, D), f32),
            pltpu.VMEM_SHARED((NPAD, H), f32),
            pltpu.SemaphoreType.DMA((3, 2)),
            pltpu.SemaphoreType.DMA((2, 3)),
            pltpu.SemaphoreType.DMA((2, 2)),
        ],
    )
    return call(hw, es2, ed2, mges.reshape(16), mged.reshape(16),
                src, dst, psrc, pdst, ph, z128, z8, ones8)


def _k2_body(np_r, dp_r, h_r, ap_r, gp_r, wmp_r, pa_r,
             wp1_r, bp1_r, wp2_r, bp2_r, m_r,
             zsn_r, zmn_r, zsc_r):
    m = m_r[...]
    num = np_r[0] + np_r[1]
    den = dp_r[0] + dp_r[1]
    rden = 1.0 / (den + 1e-9)
    zsc = _elu(num * jnp.dot(rden, m, preferred_element_type=jnp.float32)
               + h_r[...])
    agg = ap_r[0] + ap_r[1]
    deg = gp_r[0] + gp_r[1]
    invd = 1.0 / jnp.maximum(deg, 1.0)
    aggn = agg * jnp.dot(invd, m, preferred_element_type=jnp.float32)
    zmp = jnp.dot(aggn, wmp_r[...], preferred_element_type=jnp.float32)
    pa = pa_r[0, 0]
    zmp = jnp.where(zmp > 0, zmp, pa * zmp)

    wp1 = wp1_r[...]
    bp1 = bp1_r[...]
    wp2 = wp2_r[...]
    bp2 = bp2_r[...]

    def proj(z):
        t = _elu(jnp.dot(z, wp1, preferred_element_type=jnp.float32) + bp1)
        return jnp.dot(t, wp2, preferred_element_type=jnp.float32) + bp2

    zs = proj(zsc)
    zm = proj(zmp)
    zsn = zs / (jnp.sqrt(jnp.sum(zs * zs, axis=1, keepdims=True)) + 1e-8)
    zmn = zm / (jnp.sqrt(jnp.sum(zm * zm, axis=1, keepdims=True)) + 1e-8)
    zsn_r[...] = zsn
    zmn_r[...] = zmn
    zsc_r[...] = zsc


def _k2(num_p, den_p, h, agg_p, deg_p, w_mp, pa, wp1, bp1, wp2, bp2, m):
    f32 = jnp.float32
    return pl.pallas_call(
        _k2_body,
        grid=(NBLK,),
        in_specs=[
            pl.BlockSpec((2, ROWS, D), lambda i: (0, i, 0)),
            pl.BlockSpec((2, ROWS, H), lambda i: (0, i, 0)),
            pl.BlockSpec((ROWS, D), lambda i: (i, 0)),
            pl.BlockSpec((2, ROWS, D), lambda i: (0, i, 0)),
            pl.BlockSpec((2, ROWS, H), lambda i: (0, i, 0)),
            pl.BlockSpec((D, D), lambda i: (0, 0)),
            pl.BlockSpec((1, 1), lambda i: (0, 0)),
            pl.BlockSpec((D, D), lambda i: (0, 0)),
            pl.BlockSpec((1, D), lambda i: (0, 0)),
            pl.BlockSpec((D, D), lambda i: (0, 0)),
            pl.BlockSpec((1, D), lambda i: (0, 0)),
            pl.BlockSpec((H, D), lambda i: (0, 0)),
        ],
        out_specs=[
            pl.BlockSpec((ROWS, D), lambda i: (i, 0)),
            pl.BlockSpec((ROWS, D), lambda i: (i, 0)),
            pl.BlockSpec((ROWS, D), lambda i: (i, 0)),
        ],
        out_shape=[
            jax.ShapeDtypeStruct((N, D), f32),
            jax.ShapeDtypeStruct((N, D), f32),
            jax.ShapeDtypeStruct((N, D), f32),
        ],
    )(num_p, den_p, h, agg_p, deg_p, w_mp, pa, wp1, bp1, wp2, bp2, m)


def _k3_body(zsnb_r, zmnb_r, zsn_r, zmn_r, pos_r, zscb_r, wp_r, bp_r,
             sums_r, loss_r, out_r):
    i = pl.program_id(0)

    @pl.when(i == 0)
    def _():
        sums_r[...] = jnp.zeros((B, 4), jnp.float32)
        loss_r[...] = jnp.zeros((1, 1), jnp.float32)
        out_r[...] = (jnp.dot(zscb_r[...], wp_r[...],
                              preferred_element_type=jnp.float32) + bp_r[...])

    zsnb = zsnb_r[...]
    zmnb = zmnb_r[...]
    zsn = zsn_r[...]
    zmn = zmn_r[...]
    posf = pos_r[...].astype(jnp.float32)
    col = lax.broadcasted_iota(jnp.int32, (B, SBLK), 1) + i * SBLK
    valid = col < N
    dn = (((1,), (1,)), ((), ()))
    ssc = jnp.exp(lax.dot_general(zsnb, zmn, dn,
                                  preferred_element_type=jnp.float32) / TAU)
    ssc = jnp.where(valid, ssc, 0.0)
    smp = jnp.exp(lax.dot_general(zmnb, zsn, dn,
                                  preferred_element_type=jnp.float32) / TAU)
    smp = jnp.where(valid, smp, 0.0)
    contrib = jnp.concatenate([
        jnp.sum(ssc * posf, axis=1, keepdims=True),
        jnp.sum(ssc, axis=1, keepdims=True),
        jnp.sum(smp * posf, axis=1, keepdims=True),
        jnp.sum(smp, axis=1, keepdims=True),
    ], axis=1)
    sums_r[...] = sums_r[...] + contrib

    @pl.when(i == NSBLK - 1)
    def _():
        v = sums_r[...]
        lsc = -jnp.log(v[:, 0:1] / (v[:, 1:2] + 1e-9) + 1e-9)
        lmp = -jnp.log(v[:, 2:3] / (v[:, 3:4] + 1e-9) + 1e-9)
        msc = jnp.sum(lsc, axis=0, keepdims=True) / B
        mmp = jnp.sum(lmp, axis=0, keepdims=True) / B
        loss_r[...] = LAM * msc + (1.0 - LAM) * mmp


def _k3(zsn, zmn, pos, zsc, w_pred, b_pred):
    f32 = jnp.float32
    return pl.pallas_call(
        _k3_body,
        grid=(NSBLK,),
        in_specs=[
            pl.BlockSpec((B, D), lambda i: (0, 0)),
            pl.BlockSpec((B, D), lambda i: (0, 0)),
            pl.BlockSpec((SBLK, D), lambda i: (i, 0)),
            pl.BlockSpec((SBLK, D), lambda i: (i, 0)),
            pl.BlockSpec((B, SBLK), lambda i: (0, i)),
            pl.BlockSpec((B, D), lambda i: (0, 0)),
            pl.BlockSpec((D, OUT), lambda i: (0, 0)),
            pl.BlockSpec((1, OUT), lambda i: (0, 0)),
        ],
        out_specs=[
            pl.BlockSpec((B, 4), lambda i: (0, 0)),
            pl.BlockSpec((1, 1), lambda i: (0, 0)),
            pl.BlockSpec((B, OUT), lambda i: (0, 0)),
        ],
        out_shape=[
            jax.ShapeDtypeStruct((B, 4), f32),
            jax.ShapeDtypeStruct((1, 1), f32),
            jax.ShapeDtypeStruct((B, OUT), f32),
        ],
    )(zsn, zmn, zsn, zmn, pos, zsc, w_pred, b_pred)


def kernel(feat, edge_index, pos_feat, pos_edge_index, pos,
           W_fc, b_fc, W_sc, a_src, a_dst, W_mp, prelu_a,
           Wp1, bp1, Wp2, bp2, W_pred, b_pred):
    f32 = jnp.float32
    headsel = (jnp.arange(D)[:, None] // DH) == jnp.arange(H)[None, :]
    a2 = jnp.where(headsel, a_src.reshape(D)[:, None], 0.0)
    ad2 = jnp.where(headsel, a_dst.reshape(D)[:, None], 0.0)
    m16 = jnp.where(
        (jnp.arange(D)[None, :] // DH) == jnp.arange(H)[:, None],
        1.0, 0.0).astype(f32)

    h, ph, hw, es2, ed2, mges, mged = _k1(
        feat, pos_feat, W_fc, b_fc.reshape(1, D), W_sc, a2, ad2)

    z128 = jnp.zeros((ROWS_PER_TILE, D), f32)
    z8 = jnp.zeros((ROWS_PER_TILE, H), f32)
    ones8 = jnp.ones((CH, H), f32)

    num_p, den_p, agg_p, deg_p = _sc_edges(
        hw, es2, ed2, mges, mged, edge_index[0], edge_index[1],
        pos_edge_index[0], pos_edge_index[1], ph, z128, z8, ones8)

    zsn, zmn, zsc = _k2(num_p, den_p, h, agg_p, deg_p, W_mp,
                        prelu_a.reshape(1, 1), Wp1, bp1.reshape(1, D),
                        Wp2, bp2.reshape(1, D), m16)

    pos8 = pos.astype(jnp.int8)
    _, loss2d, out = _k3(zsn, zmn, pos8, zsc, W_pred, b_pred.reshape(1, OUT))
    return loss2d.reshape(()), out
